# baseline (speedup 1.0000x reference)
import functools

import jax
import jax.numpy as jnp
from jax import lax
from jax.experimental import pallas as pl
from jax.experimental.pallas import tpu as pltpu
from jax.experimental.pallas import tpu_sc as plsc

_L = 16
_NC = 2
_NS = 16
_NW = _NC * _NS


def _make_sc_nn(n, k):
    qpw = n // _NW
    ng = qpw // _L
    mesh = plsc.VectorSubcoreMesh(core_axis_name="c", subcore_axis_name="s")

    @functools.partial(
        pl.kernel,
        out_type=[
            jax.ShapeDtypeStruct((n,), jnp.float32),
            jax.ShapeDtypeStruct((n,), jnp.float32),
        ],
        mesh=mesh,
        compiler_params=pltpu.CompilerParams(needs_layout_passes=False),
        scratch_types=[
            pltpu.VMEM((2 * k,), jnp.float32),
            pltpu.VMEM((2 * k,), jnp.float32),
            pltpu.VMEM((k,), jnp.float32),
            pltpu.VMEM((k,), jnp.float32),
            pltpu.VMEM((k,), jnp.float32),
            pltpu.VMEM((k,), jnp.float32),
            pltpu.VMEM((2 * qpw,), jnp.float32),
            pltpu.VMEM((2 * qpw,), jnp.float32),
            pltpu.VMEM((qpw,), jnp.float32),
            pltpu.VMEM((qpw,), jnp.float32),
        ],
    )
    def sc_nn(of_h, nf_h, rf_h, pf_h, bf_h,
              d2_h, mk_h,
              rf_v, pf_v, bf_v, a_v, rx2_v, ry2_v, q2_v, n2_v, od_v, om_v):
        wid = lax.axis_index("s") * _NC + lax.axis_index("c")
        base = wid * qpw
        pltpu.sync_copy(rf_h, rf_v)
        pltpu.sync_copy(pf_h, pf_v)
        pltpu.sync_copy(bf_h, bf_v)
        pltpu.sync_copy(of_h.at[pl.ds(2 * base, 2 * qpw)], q2_v)
        pltpu.sync_copy(nf_h.at[pl.ds(2 * base, 2 * qpw)], n2_v)

        iota2 = lax.iota(jnp.int32, _L) * 2

        def pre(i, _):
            idx = iota2 + i * (2 * _L)
            rx = plsc.load_gather(rf_v, [idx])
            ry = plsc.load_gather(rf_v, [idx + 1])
            a_v[pl.ds(i * _L, _L)] = rx * rx + ry * ry
            rx2_v[pl.ds(i * _L, _L)] = rx + rx
            ry2_v[pl.ds(i * _L, _L)] = ry + ry
            return 0

        lax.fori_loop(0, k // _L, pre, 0)

        qx = [plsc.load_gather(q2_v, [iota2 + g * (2 * _L)]) for g in range(ng)]
        qy = [plsc.load_gather(q2_v, [iota2 + g * (2 * _L) + 1]) for g in range(ng)]
        inf = jnp.full((_L,), jnp.inf, jnp.float32)
        zero = jnp.zeros((_L,), jnp.int32)
        init = tuple([inf] * ng + [zero] * ng + [zero])

        def step(_, carry):
            st = list(carry)
            kv = st[2 * ng]
            ab = plsc.load_gather(a_v, [kv])
            xb = plsc.load_gather(rx2_v, [kv])
            yb = plsc.load_gather(ry2_v, [kv])
            for g in range(ng):
                t = ab - xb * qx[g] - yb * qy[g]
                pred = t < st[g]
                st[g] = jnp.where(pred, t, st[g])
                st[ng + g] = jnp.where(pred, kv, st[ng + g])
            st[2 * ng] = kv + 1
            return tuple(st)

        fin = lax.fori_loop(0, k, step, init, unroll=8)
        for g in range(ng):
            bid = fin[ng + g]
            bid2 = bid + bid
            pxg = plsc.load_gather(pf_v, [bid2])
            pyg = plsc.load_gather(pf_v, [bid2 + 1])
            bfg = plsc.load_gather(bf_v, [bid])
            nxg = plsc.load_gather(n2_v, [iota2 + g * (2 * _L)])
            nyg = plsc.load_gather(n2_v, [iota2 + g * (2 * _L) + 1])
            ddx = nxg - pxg
            ddy = nyg - pyg
            od_v[pl.ds(g * _L, _L)] = ddx * ddx + ddy * ddy
            om_v[pl.ds(g * _L, _L)] = bfg
        pltpu.sync_copy(od_v, d2_h.at[pl.ds(base, qpw)])
        pltpu.sync_copy(om_v, mk_h.at[pl.ds(base, qpw)])

    return sc_nn


def _finish_body(d2_ref, mk_ref, out_ref):
    d = jnp.sqrt(d2_ref[...])
    m = mk_ref[...]
    out_ref[0, 0] = jnp.sum(d * m) / jnp.sum(m)


def _make_finish():
    return pl.pallas_call(
        _finish_body,
        out_shape=jax.ShapeDtypeStruct((1, 1), jnp.float32),
        out_specs=pl.BlockSpec(memory_space=pltpu.SMEM),
    )


def kernel(flat_origin_curves, flat_new_curves, dict_points, dict_ref, dict_bool):
    n = flat_origin_curves.shape[0]
    k = dict_ref.shape[0]
    of = flat_origin_curves.reshape(-1)
    nf = flat_new_curves.reshape(-1)
    rf = dict_ref.reshape(-1)
    pf = dict_points.reshape(-1)
    bf = dict_bool.astype(jnp.float32)
    d2, mk = _make_sc_nn(n, k)(of, nf, rf, pf, bf)
    loss = _make_finish()(d2.reshape(n // 128, 128), mk.reshape(n // 128, 128))
    return loss[0, 0]


# speedup vs baseline: 2.8855x; 2.8855x over previous
import functools

import jax
import jax.numpy as jnp
from jax import lax
from jax.experimental import pallas as pl
from jax.experimental.pallas import tpu as pltpu
from jax.experimental.pallas import tpu_sc as plsc

_L = 16
_NC = 2
_NS = 16
_NW = _NC * _NS


def _make_sc_nn(n, k):
    qpw = n // _NW
    ng = qpw // _L
    mesh = plsc.VectorSubcoreMesh(core_axis_name="c", subcore_axis_name="s")

    @functools.partial(
        pl.kernel,
        out_type=[
            jax.ShapeDtypeStruct((n,), jnp.float32),
            jax.ShapeDtypeStruct((n,), jnp.float32),
        ],
        mesh=mesh,
        compiler_params=pltpu.CompilerParams(needs_layout_passes=False),
        scratch_types=[
            pltpu.VMEM((2 * k,), jnp.float32),
            pltpu.VMEM((2 * k,), jnp.float32),
            pltpu.VMEM((k,), jnp.float32),
            pltpu.VMEM((k,), jnp.float32),
            pltpu.VMEM((k,), jnp.float32),
            pltpu.VMEM((k,), jnp.float32),
            pltpu.VMEM((2 * qpw,), jnp.float32),
            pltpu.VMEM((2 * qpw,), jnp.float32),
            pltpu.VMEM((qpw,), jnp.float32),
            pltpu.VMEM((qpw,), jnp.float32),
        ],
    )
    def sc_nn(of_h, nf_h, rf_h, pf_h, bf_h,
              d2_h, mk_h,
              rf_v, pf_v, bf_v, a_v, rx2_v, ry2_v, q2_v, n2_v, od_v, om_v):
        wid = lax.axis_index("s") * _NC + lax.axis_index("c")
        base = wid * qpw
        pltpu.sync_copy(rf_h, rf_v)
        pltpu.sync_copy(pf_h, pf_v)
        pltpu.sync_copy(bf_h, bf_v)
        pltpu.sync_copy(of_h.at[pl.ds(2 * base, 2 * qpw)], q2_v)
        pltpu.sync_copy(nf_h.at[pl.ds(2 * base, 2 * qpw)], n2_v)

        iota2 = lax.iota(jnp.int32, _L) * 2

        def pre(i, _):
            idx = iota2 + i * (2 * _L)
            rx = plsc.load_gather(rf_v, [idx])
            ry = plsc.load_gather(rf_v, [idx + 1])
            a_v[pl.ds(i * _L, _L)] = rx * rx + ry * ry
            rx2_v[pl.ds(i * _L, _L)] = rx + rx
            ry2_v[pl.ds(i * _L, _L)] = ry + ry
            return 0

        lax.fori_loop(0, k // _L, pre, 0)

        qx = [plsc.load_gather(q2_v, [iota2 + g * (2 * _L)]) for g in range(ng)]
        qy = [plsc.load_gather(q2_v, [iota2 + g * (2 * _L) + 1]) for g in range(ng)]
        inf = jnp.full((_L,), jnp.inf, jnp.float32)
        zero = jnp.zeros((_L,), jnp.int32)
        init = tuple([inf] * ng + [zero] * ng + [zero])

        def step(_, carry):
            st = list(carry)
            kv = st[2 * ng]
            ab = plsc.load_gather(a_v, [kv])
            xb = plsc.load_gather(rx2_v, [kv])
            yb = plsc.load_gather(ry2_v, [kv])
            for g in range(ng):
                t = ab - xb * qx[g] - yb * qy[g]
                pred = t < st[g]
                st[g] = jnp.where(pred, t, st[g])
                st[ng + g] = jnp.where(pred, kv, st[ng + g])
            st[2 * ng] = kv + 1
            return tuple(st)

        fin = lax.fori_loop(0, k, step, init, unroll=1)
        for g in range(ng):
            bid = fin[ng + g]
            bid2 = bid + bid
            pxg = plsc.load_gather(pf_v, [bid2])
            pyg = plsc.load_gather(pf_v, [bid2 + 1])
            bfg = plsc.load_gather(bf_v, [bid])
            nxg = plsc.load_gather(n2_v, [iota2 + g * (2 * _L)])
            nyg = plsc.load_gather(n2_v, [iota2 + g * (2 * _L) + 1])
            ddx = nxg - pxg
            ddy = nyg - pyg
            od_v[pl.ds(g * _L, _L)] = ddx * ddx + ddy * ddy
            om_v[pl.ds(g * _L, _L)] = bfg
        pltpu.sync_copy(od_v, d2_h.at[pl.ds(base, qpw)])
        pltpu.sync_copy(om_v, mk_h.at[pl.ds(base, qpw)])

    return sc_nn


def _finish_body(d2_ref, mk_ref, out_ref):
    d = jnp.sqrt(d2_ref[...])
    m = mk_ref[...]
    out_ref[0, 0] = jnp.sum(d * m) / jnp.sum(m)


def _make_finish():
    return pl.pallas_call(
        _finish_body,
        out_shape=jax.ShapeDtypeStruct((1, 1), jnp.float32),
        out_specs=pl.BlockSpec(memory_space=pltpu.SMEM),
    )


def kernel(flat_origin_curves, flat_new_curves, dict_points, dict_ref, dict_bool):
    n = flat_origin_curves.shape[0]
    k = dict_ref.shape[0]
    of = flat_origin_curves.reshape(-1)
    nf = flat_new_curves.reshape(-1)
    rf = dict_ref.reshape(-1)
    pf = dict_points.reshape(-1)
    bf = dict_bool.astype(jnp.float32)
    d2, mk = _make_sc_nn(n, k)(of, nf, rf, pf, bf)
    loss = _make_finish()(d2.reshape(n // 128, 128), mk.reshape(n // 128, 128))
    return loss[0, 0]
